# baseline (speedup 1.0000x reference)
import functools

import jax
import jax.numpy as jnp
from jax import lax
from jax.experimental import pallas as pl
from jax.experimental.pallas import tpu as pltpu
from jax.experimental.pallas import tpu_sc as plsc

N_ROWS = 100000
N_COLS = 128
NC = 2
NS = 16
L = 16
GROUPS = N_COLS // L

B_TC = 256
K_TC = 16
ROUNDS_TC = 19
R_TC = K_TC * B_TC * ROUNDS_TC

R_SC = N_ROWS - R_TC
CHUNK = 99
NBUF = 4
ROWS_PER_CORE = R_SC // NC
CHUNKS_PER_W = ROWS_PER_CORE // (NS * CHUNK)

assert ROWS_PER_CORE % (NS * CHUNK) == 0
assert CHUNK % 9 == 0

_mesh = plsc.VectorSubcoreMesh(core_axis_name="c", subcore_axis_name="s")


@functools.partial(
    pl.kernel,
    out_type=jax.ShapeDtypeStruct((NC, N_COLS), jnp.float32),
    mesh=_mesh,
    scratch_types=[
        pltpu.VMEM((NBUF, CHUNK, N_COLS), jnp.float32),
        pltpu.VMEM((NS, N_COLS), jnp.float32),
        pltpu.VMEM((N_COLS,), jnp.float32),
        pltpu.VMEM_SHARED((NS, N_COLS), jnp.float32),
        pltpu.SemaphoreType.DMA,
        pltpu.SemaphoreType.DMA,
        pltpu.SemaphoreType.DMA,
        pltpu.SemaphoreType.DMA,
    ],
    compiler_params=pltpu.CompilerParams(use_tc_tiling_on_sc=False),
)
def _readout_sc(x_hbm, out_hbm, bufs, red_v, acc_v, shared, *sems):
    c = lax.axis_index("c")
    s = lax.axis_index("s")

    def src(j):
        row0 = R_TC + c * ROWS_PER_CORE + (j * NS + s) * CHUNK
        return x_hbm.at[pl.ds(row0, CHUNK), :]

    accs = tuple(jnp.zeros((L,), jnp.float32) for _ in range(GROUPS))

    descs = [None] * NBUF
    for j in range(min(NBUF, CHUNKS_PER_W)):
        descs[j] = pltpu.async_copy(src(j), bufs.at[j], sems[j])
    for j in range(CHUNKS_PER_W):
        b = j % NBUF
        descs[b].wait()

        def body(r, a, _b=b):
            return tuple(a[g] + bufs[_b, r, g * L:(g + 1) * L]
                         for g in range(GROUPS))

        accs = plsc.parallel_loop(0, CHUNK, unroll=9, carry=accs)(body)
        if j + NBUF < CHUNKS_PER_W:
            descs[b] = pltpu.async_copy(src(j + NBUF), bufs.at[b], sems[b])

    for g in range(GROUPS):
        acc_v[pl.ds(g * L, L)] = accs[g]
    pltpu.sync_copy(acc_v, shared.at[s])
    plsc.subcore_barrier()

    @pl.when(s == 0)
    def _():
        pltpu.sync_copy(shared, red_v)

        def body2(i, a):
            return tuple(a[g] + red_v[i, g * L:(g + 1) * L]
                         for g in range(GROUPS))

        faccs = lax.fori_loop(
            0, NS, body2,
            tuple(jnp.zeros((L,), jnp.float32) for _ in range(GROUPS)))
        for g in range(GROUPS):
            acc_v[pl.ds(g * L, L)] = faccs[g]
        pltpu.sync_copy(acc_v, out_hbm.at[c])


def _tc_body(x_hbm, o_ref, bufs, acc_ref, sems):
    def chunk(i):
        return x_hbm.at[pl.ds(i * B_TC, B_TC), :]

    acc_ref[...] = jnp.zeros_like(acc_ref)
    for j in range(K_TC):
        pltpu.async_copy(chunk(j), bufs.at[j], sems.at[j])

    def round_body(r, _):
        for j in range(K_TC):
            pltpu.make_async_copy(chunk(r * K_TC + j), bufs.at[j],
                                  sems.at[j]).wait()
            acc_ref[...] += jnp.sum(
                bufs[j].reshape(B_TC // 8, 8, N_COLS), axis=0)

            @pl.when(r < ROUNDS_TC - 1)
            def _():
                pltpu.async_copy(chunk((r + 1) * K_TC + j), bufs.at[j],
                                 sems.at[j])
        return 0

    lax.fori_loop(0, ROUNDS_TC, round_body, 0)
    o_ref[...] = jnp.sum(acc_ref[...], axis=0, keepdims=True)


_tc_call = pl.pallas_call(
    _tc_body,
    in_specs=[pl.BlockSpec(memory_space=pl.ANY)],
    out_shape=jax.ShapeDtypeStruct((1, N_COLS), jnp.float32),
    scratch_shapes=[
        pltpu.VMEM((K_TC, B_TC, N_COLS), jnp.float32),
        pltpu.VMEM((8, N_COLS), jnp.float32),
        pltpu.SemaphoreType.DMA((K_TC,)),
    ],
)


def kernel(x):
    sc_part = _readout_sc(x)
    tc_part = _tc_call(x)
    return tc_part + sc_part[0:1] + sc_part[1:2]


# speedup vs baseline: 1.7101x; 1.7101x over previous
import jax
import jax.numpy as jnp
from jax.experimental import pallas as pl
from jax.experimental.pallas import tpu as pltpu

N_ROWS = 100000
N_COLS = 128

B_TC = 400
NSTREAM = 10
G_TC = 25
assert NSTREAM * B_TC * G_TC == N_ROWS


def _tc_body(*refs):
    x_refs = refs[:NSTREAM]
    o_ref = refs[NSTREAM]
    acc_ref = refs[NSTREAM + 1]
    i = pl.program_id(0)

    @pl.when(i == 0)
    def _():
        acc_ref[...] = jnp.zeros_like(acc_ref)

    part = acc_ref[...]
    for x_ref in x_refs:
        part += jnp.sum(x_ref[...].reshape(B_TC // 8, 8, N_COLS), axis=0)
    acc_ref[...] = part

    @pl.when(i == G_TC - 1)
    def _():
        o_ref[...] = jnp.sum(acc_ref[...], axis=0, keepdims=True)


_tc_call = pl.pallas_call(
    _tc_body,
    grid=(G_TC,),
    in_specs=[
        pl.BlockSpec((B_TC, N_COLS), lambda i, _k=k: (i * NSTREAM + _k, 0))
        for k in range(NSTREAM)
    ],
    out_specs=pl.BlockSpec((1, N_COLS), lambda i: (0, 0)),
    out_shape=jax.ShapeDtypeStruct((1, N_COLS), jnp.float32),
    scratch_shapes=[pltpu.VMEM((8, N_COLS), jnp.float32)],
)


def kernel(x):
    return _tc_call(*([x] * NSTREAM))
